# baseline (speedup 1.0000x reference)
import functools

import jax
import jax.numpy as jnp
from jax import lax
from jax.experimental import pallas as pl
from jax.experimental.pallas import tpu as pltpu
from jax.experimental.pallas import tpu_sc as plsc

_L = 16
_NW = 32
_CR = 8
_SC_ROWS = 10240
_TC_BLOCK = 512


def _sc_gather(x, inds, n_rows):
    N, F = x.shape
    K = inds.shape[0]
    rows = n_rows // _NW
    NCH = rows // _CR
    L = _L
    CR = _CR

    mesh = plsc.VectorSubcoreMesh(core_axis_name="c", subcore_axis_name="s")

    @functools.partial(
        pl.kernel,
        out_type=jax.ShapeDtypeStruct((n_rows, K), jnp.float32),
        mesh=mesh,
        scratch_types=[
            pltpu.VMEM((3 * CR, F), jnp.float32),
            pltpu.VMEM((2, CR, K), jnp.float32),
            pltpu.VMEM((K,), jnp.int32),
            pltpu.SemaphoreType.DMA,
            pltpu.SemaphoreType.DMA,
        ],
        compiler_params=pltpu.CompilerParams(
            use_tc_tiling_on_sc=True, needs_layout_passes=False
        ),
    )
    def gather_cols(x_hbm, inds_hbm, out_hbm, inbuf, outbuf, indsbuf, insem, outsem):
        wid = lax.axis_index("s") * 2 + lax.axis_index("c")
        base = wid * rows
        pltpu.sync_copy(inds_hbm, indsbuf)
        cols = [indsbuf[pl.ds(L * j, L)] for j in range(K // L)]

        def in_cp(i, slot):
            return pltpu.make_async_copy(
                x_hbm.at[pl.ds(base + i * CR, CR)],
                inbuf.at[pl.ds(slot * CR, CR)],
                insem,
            )

        def out_cp(i, oslot):
            return pltpu.make_async_copy(
                outbuf.at[oslot],
                out_hbm.at[pl.ds(base + i * CR, CR)],
                outsem,
            )

        in_cp(0, 0).start()
        in_cp(1, 1).start()

        def chunk_body(i, carry):
            slot = lax.rem(i, 3)
            oslot = lax.rem(i, 2)
            in_cp(i, slot).wait()

            @pl.when(i + 2 < NCH)
            def _():
                in_cp(i + 2, lax.rem(i + 2, 3)).start()

            @pl.when(i >= 2)
            def _():
                out_cp(i - 2, oslot).wait()

            for r in range(CR):
                rowv = jnp.zeros((L,), jnp.int32) + (slot * CR + r)
                for j in range(K // L):
                    outbuf[oslot, r, pl.ds(L * j, L)] = plsc.load_gather(
                        inbuf, [rowv, cols[j]]
                    )
            out_cp(i, oslot).start()
            return carry

        lax.fori_loop(0, NCH, chunk_body, 0)
        out_cp(NCH - 2, lax.rem(NCH - 2, 2)).wait()
        out_cp(NCH - 1, lax.rem(NCH - 1, 2)).wait()

    return gather_cols(x, inds)


def _tc_gather(x, inds, row0):
    N, F = x.shape
    K = inds.shape[0]
    n_tc = N - row0
    blk0 = row0 // _TC_BLOCK

    def body(x_ref, inds_ref, out_ref):
        col = lax.broadcasted_iota(jnp.int32, (F, K), 0)
        sel = jnp.broadcast_to(inds_ref[...][None, :], (F, K))
        mask = (col == sel).astype(jnp.float32)
        out_ref[...] = jnp.dot(
            x_ref[...],
            mask,
            preferred_element_type=jnp.float32,
            precision=lax.Precision.HIGHEST,
        )

    return pl.pallas_call(
        body,
        grid=(n_tc // _TC_BLOCK,),
        in_specs=[
            pl.BlockSpec((_TC_BLOCK, F), lambda i: (blk0 + i, 0)),
            pl.BlockSpec((K,), lambda i: (0,)),
        ],
        out_specs=pl.BlockSpec((_TC_BLOCK, K), lambda i: (i, 0)),
        out_shape=jax.ShapeDtypeStruct((n_tc, K), jnp.float32),
    )(x, inds)


def kernel(x, inds):
    n_sc = _SC_ROWS
    out_sc = _sc_gather(x, inds, n_sc)
    out_tc = _tc_gather(x, inds, n_sc)
    return jnp.concatenate([out_sc, out_tc], axis=0)


# speedup vs baseline: 1.0154x; 1.0154x over previous
import functools

import jax
import jax.numpy as jnp
from jax import lax
from jax.experimental import pallas as pl
from jax.experimental.pallas import tpu as pltpu
from jax.experimental.pallas import tpu_sc as plsc


def kernel(x, inds):
    N, F = x.shape
    K = inds.shape[0]
    L = 16

    NW = 32
    rows = N // NW
    CR = 8
    NCH = rows // CR

    mesh = plsc.VectorSubcoreMesh(core_axis_name="c", subcore_axis_name="s")

    @functools.partial(
        pl.kernel,
        out_type=jax.ShapeDtypeStruct((N, K), jnp.float32),
        mesh=mesh,
        scratch_types=[
            pltpu.VMEM((3 * CR, F), jnp.float32),
            pltpu.VMEM((2, CR, K), jnp.float32),
            pltpu.VMEM((K,), jnp.int32),
            pltpu.SemaphoreType.DMA,
            pltpu.SemaphoreType.DMA,
        ],
        compiler_params=pltpu.CompilerParams(
            use_tc_tiling_on_sc=True, needs_layout_passes=False
        ),
    )
    def gather_cols(x_hbm, inds_hbm, out_hbm, inbuf, outbuf, indsbuf, insem, outsem):
        wid = lax.axis_index("s") * 2 + lax.axis_index("c")
        base = wid * rows
        pltpu.sync_copy(inds_hbm, indsbuf)
        cols = [indsbuf[pl.ds(L * j, L)] for j in range(K // L)]

        def in_cp(i, slot):
            return pltpu.make_async_copy(
                x_hbm.at[pl.ds(base + i * CR, CR)],
                inbuf.at[pl.ds(slot * CR, CR)],
                insem,
            )

        def out_cp(i, oslot):
            return pltpu.make_async_copy(
                outbuf.at[oslot],
                out_hbm.at[pl.ds(base + i * CR, CR)],
                outsem,
            )

        in_cp(0, 0).start()
        in_cp(1, 1).start()

        def chunk_body(i, carry):
            slot = lax.rem(i, 3)
            oslot = lax.rem(i, 2)

            @pl.when(i + 2 < NCH)
            def _():
                in_cp(i + 2, lax.rem(i + 2, 3)).start()

            in_cp(i, slot).wait()

            @pl.when(i >= 2)
            def _():
                out_cp(i - 2, oslot).wait()

            for r in range(CR):
                rowv = jnp.zeros((L,), jnp.int32) + (slot * CR + r)
                for j in range(K // L):
                    outbuf[oslot, r, pl.ds(L * j, L)] = plsc.load_gather(
                        inbuf, [rowv, cols[j]]
                    )
            out_cp(i, oslot).start()
            return carry

        lax.fori_loop(0, NCH, chunk_body, 0)
        out_cp(NCH - 2, lax.rem(NCH - 2, 2)).wait()
        out_cp(NCH - 1, lax.rem(NCH - 1, 2)).wait()

    return gather_cols(x, inds)
